# baseline (speedup 1.0000x reference)
import functools

import jax
import jax.numpy as jnp
import numpy as np
from jax.experimental import pallas as pl

_BASE = np.array([[1, 0], [3, 0], [5, 0], [7, 0], [9, 0], [11, 0], [0, 2], [2, 2], [4, 2], [6, 2], [8, 2], [10, 2], [12, 2], [1, 4], [3, 4], [5, 4], [7, 4], [9, 4], [11, 4], [2, 6], [4, 6], [6, 6], [8, 6], [10, 6], [3, 8], [5, 8], [7, 8], [9, 8], [4, 10], [6, 10], [8, 10], [5, 12], [7, 12]], dtype=np.int64)
_H_PAD = 13
_W_PAD = 13


def _build_weight() -> np.ndarray:
    bx = _BASE[:, 0]
    by = _BASE[:, 1]
    bxm1 = np.maximum(bx - 1, 0)
    bxp1 = np.minimum(bx + 1, _H_PAD - 1)
    bym1 = np.maximum(by - 1, 0)
    byp1 = np.minimum(by + 1, _W_PAD - 1)
    mx = bx // 2
    my = by // 2
    mx = mx + (my + 1) % 2
    w = np.zeros((_H_PAD * _W_PAD, 49), dtype=np.float32)
    taps = [(bx, by), (bxm1, by), (bxp1, by), (bx, byp1), (bx, bym1),
            (bxm1, byp1), (bxm1, bym1)]
    for tx, ty in taps:
        for i in range(len(bx)):
            w[tx[i] * _W_PAD + ty[i], mx[i] * 7 + my[i]] += 1.0 / 7.0
    return w


_W_NP = _build_weight()


def _pool_body(x_ref, w_ref, o_ref):
    o_ref[...] = jax.lax.dot(
        x_ref[...], w_ref[...], preferred_element_type=jnp.float32)


@functools.partial(jax.jit, static_argnums=())
def kernel(input):
    b, c, h, w = input.shape
    n = b * c
    x = input.reshape(n, h * w)
    blk = 4096
    wmat = jnp.asarray(_W_NP)
    out = pl.pallas_call(
        _pool_body,
        grid=(n // blk,),
        in_specs=[
            pl.BlockSpec((blk, h * w), lambda i: (i, 0)),
            pl.BlockSpec((h * w, 49), lambda i: (0, 0)),
        ],
        out_specs=pl.BlockSpec((blk, 49), lambda i: (i, 0)),
        out_shape=jax.ShapeDtypeStruct((n, 49), jnp.float32),
    )(x, wmat)
    return out.reshape(b, c, 7, 7)
